# Initial kernel scaffold; baseline (speedup 1.0000x reference)
import jax
import jax.numpy as jnp
from jax.experimental import pallas as pl


def kernel(input, w):
    raise NotImplementedError("write your pallas kernel here")


# speedup vs baseline: 6.0627x; 6.0627x over previous
import jax
import jax.numpy as jnp
from jax.experimental import pallas as pl

SIZE = 128
ROW_BLOCK = 128


def _snn_body(xt_ref, w_ref, out_ref):
    xt = xt_ref[...]
    w = w_ref[...]
    jiota = jax.lax.broadcasted_iota(jnp.int32, (SIZE, 1), 0)

    def rank_step(k, rank_t):
        xk = xt_ref[pl.ds(k, 1), :]
        lt = xk < xt
        tie = jnp.logical_and(xk == xt, k < jiota)
        return rank_t + jnp.logical_or(lt, tie).astype(jnp.float32)

    rank_t = jax.lax.fori_loop(
        0, SIZE, rank_step, jnp.zeros((SIZE, ROW_BLOCK), jnp.float32))

    zeros = jnp.zeros((SIZE, ROW_BLOCK), jnp.float32)

    def scan_step(i, carry):
        c_w, c_wx, num, den = carry
        p = (rank_t == i.astype(jnp.float32)).astype(jnp.float32)
        a = jax.lax.dot_general(
            w, p, (((1,), (0,)), ((), ())),
            precision=jax.lax.Precision.HIGHEST,
            preferred_element_type=jnp.float32)
        sx = jnp.sum(xt * p, axis=0, keepdims=True)
        c_w = c_w + a
        c_wx = c_wx + a * sx
        denom = jnp.clip(c_w - 1.0, 1e-10, 1e10)
        cond = jnp.logical_and(c_wx > sx * denom, c_w > 1.0)
        newly = jnp.logical_and(cond, den == 0.0)
        num = jnp.where(newly, c_wx, num)
        den = jnp.where(newly, denom, den)
        return c_w, c_wx, num, den

    _, _, num, den = jax.lax.fori_loop(
        0, SIZE, scan_step, (zeros, zeros, zeros, zeros))
    out_ref[...] = jnp.where(den == 0.0, 1e10, num / den)


@jax.jit
def kernel(input, w):
    x = input
    batch = x.shape[0]
    out = pl.pallas_call(
        _snn_body,
        grid=(batch // ROW_BLOCK,),
        in_specs=[
            pl.BlockSpec((SIZE, ROW_BLOCK), lambda g: (0, g)),
            pl.BlockSpec((SIZE, SIZE), lambda g: (0, 0)),
        ],
        out_specs=pl.BlockSpec((SIZE, ROW_BLOCK), lambda g: (0, g)),
        out_shape=jax.ShapeDtypeStruct((SIZE, batch), jnp.float32),
    )(x.T, w)
    return out.T[:, :, None]
